# baseline (speedup 1.0000x reference)
import functools

import jax
import jax.numpy as jnp
from jax import lax
from jax.experimental import pallas as pl
from jax.experimental.pallas import tpu as pltpu
from jax.experimental.pallas import tpu_sc as plsc

_NCLASS = 1000
_NCPAD = 1024
_D = 128
_B = 16384
_L = 16
_NC = 2
_NS = 16
_NW = _NC * _NS
_BPW = _B // _NW
_NCHUNK = 4
_CH = _BPW // _NCHUNK
_NVJ = _D // _L
_NBUF = 2
_UNROLL = 2


def _sc_center_loss(feat_r, y_r, negc):
  mesh = plsc.VectorSubcoreMesh(core_axis_name="c", subcore_axis_name="s")

  @functools.partial(
      pl.kernel,
      mesh=mesh,
      out_type=jax.ShapeDtypeStruct((_NW, _L), jnp.float32),
      scratch_types=[
          pltpu.VMEM((_NCHUNK, _CH), jnp.int32),
          pltpu.VMEM((_CH, _D), jnp.float32),
          pltpu.VMEM((_CH, _D), jnp.float32),
          pltpu.VMEM((_CH, _D), jnp.float32),
          pltpu.VMEM((_L,), jnp.float32),
          pltpu.VMEM_SHARED((_NCPAD, _D), jnp.float32),
          pltpu.SemaphoreType.DMA,
          pltpu.SemaphoreType.DMA,
          pltpu.SemaphoreType.DMA,
          pltpu.SemaphoreType.DMA,
          pltpu.SemaphoreType.DMA,
          pltpu.SemaphoreType.DMA,
      ],
  )
  def body(feat_hbm, y_hbm, negc_hbm, out_hbm,
           idx_v, b0, b1, b2, acc_v, spm,
           f0, f1, f2, a0, a1, a2):
    wid = lax.axis_index("s") * _NC + lax.axis_index("c")
    sid = lax.axis_index("s")
    pltpu.sync_copy(y_hbm.at[wid], idx_v)

    bufs = (b0, b1, b2)
    fsems = (f0, f1, f2)
    asems = (a0, a1, a2)

    feat_h = [None] * _NCHUNK
    add_h = [None] * _NCHUNK

    def fire_feat(c):
      feat_h[c] = pltpu.async_copy(
          feat_hbm.at[wid * _NCHUNK + c], bufs[c % _NBUF], fsems[c % _NBUF])

    def fire_add(c):
      feat_h[c].wait()
      add_h[c] = pltpu.async_copy(
          spm.at[idx_v.at[c]], bufs[c % _NBUF], asems[c % _NBUF],
          add=True)

    for c in range(_NBUF):
      fire_feat(c)
    rpt = _NCPAD // _NS
    pltpu.sync_copy(negc_hbm.at[pl.ds(sid * rpt, rpt)],
                    spm.at[pl.ds(sid * rpt, rpt)])
    plsc.subcore_barrier()
    fire_add(0)

    accs = tuple(jnp.zeros((_L,), jnp.float32) for _ in range(_NVJ))
    for c in range(_NCHUNK):
      if c + 1 < _NCHUNK:
        fire_add(c + 1)
      add_h[c].wait()
      buf = bufs[c % _NBUF]

      def row_body(i, acc, buf=buf):
        new = list(acc)
        base = i * _UNROLL
        for r in range(_UNROLL):
          for j in range(_NVJ):
            d = buf[base + r, pl.ds(j * _L, _L)]
            new[j] = new[j] + d * d
        return tuple(new)

      accs = lax.fori_loop(0, _CH // _UNROLL, row_body, accs)
      if c + _NBUF < _NCHUNK:
        fire_feat(c + _NBUF)

    total = accs[0]
    for j in range(1, _NVJ):
      total = total + accs[j]
    acc_v[...] = total
    pltpu.sync_copy(acc_v, out_hbm.at[wid])

  return body(feat_r, y_r, negc)


def kernel(feat, logits, y, centers):
  del logits
  y_r = y.astype(jnp.int32).reshape(_NW, _NCHUNK, _CH)
  feat_r = feat.reshape(_NW * _NCHUNK, _CH, _D)
  negc = jnp.zeros((_NCPAD, _D), jnp.float32).at[:_NCLASS].set(-centers)
  partials = _sc_center_loss(feat_r, y_r, negc)
  return jnp.sum(partials) / 2.0 / _B


# speedup vs baseline: 1.0073x; 1.0073x over previous
import functools

import jax
import jax.numpy as jnp
from jax import lax
from jax.experimental import pallas as pl
from jax.experimental.pallas import tpu as pltpu
from jax.experimental.pallas import tpu_sc as plsc

_NCLASS = 1000
_D = 128
_B = 16384
_L = 16
_NC = 2
_NS = 16
_NW = _NC * _NS
_BPW = _B // _NW
_NCHUNK = 4
_CH = _BPW // _NCHUNK
_NVJ = _D // _L
_NBUF = 4
_UNROLL = 4
_NSTAGE = 5
_RPT = _NCLASS // _NSTAGE


def _sc_center_loss(feat_r, y_r, negc):
  mesh = plsc.VectorSubcoreMesh(core_axis_name="c", subcore_axis_name="s")

  @functools.partial(
      pl.kernel,
      mesh=mesh,
      out_type=jax.ShapeDtypeStruct((_NW, _L), jnp.float32),
      scratch_types=[
          pltpu.VMEM((_NCHUNK, _CH), jnp.int32),
          pltpu.VMEM((_CH, _D), jnp.float32),
          pltpu.VMEM((_CH, _D), jnp.float32),
          pltpu.VMEM((_CH, _D), jnp.float32),
          pltpu.VMEM((_CH, _D), jnp.float32),
          pltpu.VMEM((_L,), jnp.float32),
          pltpu.VMEM_SHARED((_NCLASS, _D), jnp.float32),
          pltpu.SemaphoreType.DMA,
          pltpu.SemaphoreType.DMA,
          pltpu.SemaphoreType.DMA,
          pltpu.SemaphoreType.DMA,
          pltpu.SemaphoreType.DMA,
          pltpu.SemaphoreType.DMA,
          pltpu.SemaphoreType.DMA,
          pltpu.SemaphoreType.DMA,
      ],
  )
  def body(feat_hbm, y_hbm, negc_hbm, out_hbm,
           idx_v, b0, b1, b2, b3, acc_v, spm,
           f0, f1, f2, f3, a0, a1, a2, a3):
    wid = lax.axis_index("s") * _NC + lax.axis_index("c")
    sid = lax.axis_index("s")
    pltpu.sync_copy(y_hbm.at[wid], idx_v)

    bufs = (b0, b1, b2, b3)
    fsems = (f0, f1, f2, f3)
    asems = (a0, a1, a2, a3)

    feat_h = [None] * _NCHUNK
    add_h = [None] * _NCHUNK

    def fire_feat(c):
      feat_h[c] = pltpu.async_copy(
          feat_hbm.at[wid * _NCHUNK + c], bufs[c % _NBUF], fsems[c % _NBUF])

    def fire_add(c):
      feat_h[c].wait()
      add_h[c] = pltpu.async_copy(
          spm.at[idx_v.at[c]], bufs[c % _NBUF], asems[c % _NBUF],
          add=True)

    for c in range(_NBUF):
      fire_feat(c)
    @pl.when(sid < _NSTAGE)
    def _():
      pltpu.sync_copy(negc_hbm.at[pl.ds(sid * _RPT, _RPT)],
                      spm.at[pl.ds(sid * _RPT, _RPT)])
    plsc.subcore_barrier()
    fire_add(0)

    accs = tuple(jnp.zeros((_L,), jnp.float32) for _ in range(_NVJ))
    for c in range(_NCHUNK):
      if c + 1 < _NCHUNK:
        fire_add(c + 1)
      add_h[c].wait()
      buf = bufs[c % _NBUF]

      def row_body(i, acc, buf=buf):
        new = list(acc)
        base = i * _UNROLL
        for r in range(_UNROLL):
          for j in range(_NVJ):
            d = buf[base + r, pl.ds(j * _L, _L)]
            new[j] = new[j] + d * d
        return tuple(new)

      accs = lax.fori_loop(0, _CH // _UNROLL, row_body, accs)
      if c + _NBUF < _NCHUNK:
        fire_feat(c + _NBUF)

    total = accs[0]
    for j in range(1, _NVJ):
      total = total + accs[j]
    acc_v[...] = total
    pltpu.sync_copy(acc_v, out_hbm.at[wid])

  return body(feat_r, y_r, negc)


def kernel(feat, logits, y, centers):
  del logits
  y_r = y.astype(jnp.int32).reshape(_NW, _NCHUNK, _CH)
  feat_r = feat.reshape(_NW * _NCHUNK, _CH, _D)
  partials = _sc_center_loss(feat_r, y_r, -centers)
  return jnp.sum(partials) / 2.0 / _B


# speedup vs baseline: 1.0180x; 1.0106x over previous
import functools

import jax
import jax.numpy as jnp
from jax import lax
from jax.experimental import pallas as pl
from jax.experimental.pallas import tpu as pltpu
from jax.experimental.pallas import tpu_sc as plsc

_NCLASS = 1000
_D = 128
_B = 16384
_L = 16
_NC = 2
_NS = 16
_NW = _NC * _NS
_BPW = _B // _NW
_NCHUNK = 8
_CH = _BPW // _NCHUNK
_NVJ = _D // _L
_NBUF = 4
_UNROLL = 4
_NSTAGE = 5
_RPT = _NCLASS // _NSTAGE


def _sc_center_loss(feat_r, y_r, negc):
  mesh = plsc.VectorSubcoreMesh(core_axis_name="c", subcore_axis_name="s")

  @functools.partial(
      pl.kernel,
      mesh=mesh,
      out_type=jax.ShapeDtypeStruct((_NW, _L), jnp.float32),
      scratch_types=[
          pltpu.VMEM((_NCHUNK, _CH), jnp.int32),
          pltpu.VMEM((_CH, _D), jnp.float32),
          pltpu.VMEM((_CH, _D), jnp.float32),
          pltpu.VMEM((_CH, _D), jnp.float32),
          pltpu.VMEM((_CH, _D), jnp.float32),
          pltpu.VMEM((_L,), jnp.float32),
          pltpu.VMEM_SHARED((_NCLASS, _D), jnp.float32),
          pltpu.SemaphoreType.DMA,
          pltpu.SemaphoreType.DMA,
          pltpu.SemaphoreType.DMA,
          pltpu.SemaphoreType.DMA,
          pltpu.SemaphoreType.DMA,
          pltpu.SemaphoreType.DMA,
          pltpu.SemaphoreType.DMA,
          pltpu.SemaphoreType.DMA,
      ],
  )
  def body(feat_hbm, y_hbm, negc_hbm, out_hbm,
           idx_v, b0, b1, b2, b3, acc_v, spm,
           f0, f1, f2, f3, a0, a1, a2, a3):
    wid = lax.axis_index("s") * _NC + lax.axis_index("c")
    sid = lax.axis_index("s")
    pltpu.sync_copy(y_hbm.at[wid], idx_v)

    bufs = (b0, b1, b2, b3)
    fsems = (f0, f1, f2, f3)
    asems = (a0, a1, a2, a3)

    feat_h = [None] * _NCHUNK
    add_h = [None] * _NCHUNK

    def fire_feat(c):
      feat_h[c] = pltpu.async_copy(
          feat_hbm.at[wid * _NCHUNK + c], bufs[c % _NBUF], fsems[c % _NBUF])

    def fire_add(c):
      feat_h[c].wait()
      add_h[c] = pltpu.async_copy(
          spm.at[idx_v.at[c]], bufs[c % _NBUF], asems[c % _NBUF],
          add=True)

    for c in range(_NBUF):
      fire_feat(c)
    @pl.when(sid < _NSTAGE)
    def _():
      pltpu.sync_copy(negc_hbm.at[pl.ds(sid * _RPT, _RPT)],
                      spm.at[pl.ds(sid * _RPT, _RPT)])
    plsc.subcore_barrier()
    fire_add(0)

    accs = tuple(jnp.zeros((_L,), jnp.float32) for _ in range(_NVJ))
    for c in range(_NCHUNK):
      if c + 1 < _NCHUNK:
        fire_add(c + 1)
      add_h[c].wait()
      buf = bufs[c % _NBUF]

      def row_body(i, acc, buf=buf):
        new = list(acc)
        base = i * _UNROLL
        for r in range(_UNROLL):
          for j in range(_NVJ):
            d = buf[base + r, pl.ds(j * _L, _L)]
            new[j] = new[j] + d * d
        return tuple(new)

      accs = lax.fori_loop(0, _CH // _UNROLL, row_body, accs)
      if c + _NBUF < _NCHUNK:
        fire_feat(c + _NBUF)

    total = accs[0]
    for j in range(1, _NVJ):
      total = total + accs[j]
    acc_v[...] = total
    pltpu.sync_copy(acc_v, out_hbm.at[wid])

  return body(feat_r, y_r, negc)


def kernel(feat, logits, y, centers):
  del logits
  y_r = y.astype(jnp.int32).reshape(_NW, _NCHUNK, _CH)
  feat_r = feat.reshape(_NW * _NCHUNK, _CH, _D)
  partials = _sc_center_loss(feat_r, y_r, -centers)
  return jnp.sum(partials) / 2.0 / _B


# speedup vs baseline: 1.0356x; 1.0173x over previous
import functools

import jax
import jax.numpy as jnp
from jax import lax
from jax.experimental import pallas as pl
from jax.experimental.pallas import tpu as pltpu
from jax.experimental.pallas import tpu_sc as plsc

_NCLASS = 1000
_NCPAD = 1024
_D = 128
_B = 16384
_L = 16
_NC = 2
_NS = 16
_NW = _NC * _NS
_BPW = _B // _NW
_NCHUNK = 4
_CH = _BPW // _NCHUNK
_NVJ = _D // _L
_NBUF = 3
_UNROLL = 4


def _sc_center_loss(feat_r, y_r, negc):
  mesh = plsc.VectorSubcoreMesh(core_axis_name="c", subcore_axis_name="s")

  @functools.partial(
      pl.kernel,
      mesh=mesh,
      out_type=jax.ShapeDtypeStruct((_NW, _L), jnp.float32),
      scratch_types=[
          pltpu.VMEM((_NCHUNK, _CH), jnp.int32),
          pltpu.VMEM((_CH, _D), jnp.float32),
          pltpu.VMEM((_CH, _D), jnp.float32),
          pltpu.VMEM((_CH, _D), jnp.float32),
          pltpu.VMEM((_L,), jnp.float32),
          pltpu.VMEM_SHARED((_NCPAD, _D), jnp.float32),
          pltpu.SemaphoreType.DMA,
          pltpu.SemaphoreType.DMA,
          pltpu.SemaphoreType.DMA,
          pltpu.SemaphoreType.DMA,
          pltpu.SemaphoreType.DMA,
          pltpu.SemaphoreType.DMA,
      ],
  )
  def body(feat_hbm, y_hbm, negc_hbm, out_hbm,
           idx_v, b0, b1, b2, acc_v, spm,
           f0, f1, f2, a0, a1, a2):
    wid = lax.axis_index("s") * _NC + lax.axis_index("c")
    sid = lax.axis_index("s")

    bufs = (b0, b1, b2)
    fsems = (f0, f1, f2)
    asems = (a0, a1, a2)

    feat_h = [None] * _NCHUNK
    add_h = [None] * _NCHUNK

    def fire_feat(c):
      feat_h[c] = pltpu.async_copy(
          feat_hbm.at[wid * _NCHUNK + c], bufs[c % _NBUF], fsems[c % _NBUF])

    def fire_add(c):
      feat_h[c].wait()
      add_h[c] = pltpu.async_copy(
          spm.at[idx_v.at[c]], bufs[c % _NBUF], asems[c % _NBUF],
          add=True)

    for c in range(_NBUF):
      fire_feat(c)
    pltpu.sync_copy(y_hbm.at[wid], idx_v)
    rpt = _NCPAD // _NS
    pltpu.sync_copy(negc_hbm.at[pl.ds(sid * rpt, rpt)],
                    spm.at[pl.ds(sid * rpt, rpt)])
    plsc.subcore_barrier()
    fire_add(0)

    accs = tuple(jnp.zeros((_L,), jnp.float32) for _ in range(_NVJ))
    for c in range(_NCHUNK):
      if c + 1 < _NCHUNK:
        fire_add(c + 1)
      add_h[c].wait()
      buf = bufs[c % _NBUF]

      def row_body(i, acc, buf=buf):
        new = list(acc)
        base = i * _UNROLL
        for r in range(_UNROLL):
          for j in range(_NVJ):
            d = buf[base + r, pl.ds(j * _L, _L)]
            new[j] = new[j] + d * d
        return tuple(new)

      accs = lax.fori_loop(0, _CH // _UNROLL, row_body, accs)
      if c + _NBUF < _NCHUNK:
        fire_feat(c + _NBUF)

    total = accs[0]
    for j in range(1, _NVJ):
      total = total + accs[j]
    acc_v[...] = total
    pltpu.sync_copy(acc_v, out_hbm.at[wid])

  return body(feat_r, y_r, negc)


def kernel(feat, logits, y, centers):
  del logits
  y_r = y.astype(jnp.int32).reshape(_NW, _NCHUNK, _CH)
  feat_r = feat.reshape(_NW * _NCHUNK, _CH, _D)
  negc = jnp.zeros((_NCPAD, _D), jnp.float32).at[:_NCLASS].set(-centers)
  partials = _sc_center_loss(feat_r, y_r, negc)
  return jnp.sum(partials) / 2.0 / _B


# speedup vs baseline: 1.0405x; 1.0047x over previous
import functools

import jax
import jax.numpy as jnp
from jax import lax
from jax.experimental import pallas as pl
from jax.experimental.pallas import tpu as pltpu
from jax.experimental.pallas import tpu_sc as plsc

_NCLASS = 1000
_NCPAD = 1024
_D = 128
_B = 16384
_L = 16
_NC = 2
_NS = 16
_NW = _NC * _NS
_BPW = _B // _NW
_NCHUNK = 4
_CH = _BPW // _NCHUNK
_NVJ = _D // _L
_NBUF = 3
_UNROLL = 4


def _sc_center_loss(feat_r, y_r, negc):
  mesh = plsc.VectorSubcoreMesh(core_axis_name="c", subcore_axis_name="s")

  @functools.partial(
      pl.kernel,
      mesh=mesh,
      out_type=jax.ShapeDtypeStruct((_NW, _L), jnp.float32),
      scratch_types=[
          pltpu.VMEM((_NCHUNK, _CH), jnp.int32),
          pltpu.VMEM((_CH, _D), jnp.float32),
          pltpu.VMEM((_CH, _D), jnp.float32),
          pltpu.VMEM((_CH, _D), jnp.float32),
          pltpu.VMEM((_L,), jnp.float32),
          pltpu.VMEM_SHARED((_NCPAD, _D), jnp.float32),
          pltpu.SemaphoreType.DMA,
          pltpu.SemaphoreType.DMA,
          pltpu.SemaphoreType.DMA,
          pltpu.SemaphoreType.DMA,
          pltpu.SemaphoreType.DMA,
          pltpu.SemaphoreType.DMA,
          pltpu.SemaphoreType.DMA,
      ],
  )
  def body(feat_hbm, y_hbm, negc_hbm, out_hbm,
           idx_v, b0, b1, b2, acc_v, spm,
           f0, f1, f2, a0, a1, a2, ssem):
    wid = lax.axis_index("s") * _NC + lax.axis_index("c")
    sid = lax.axis_index("s")

    bufs = (b0, b1, b2)
    fsems = (f0, f1, f2)
    asems = (a0, a1, a2)

    feat_h = [None] * _NCHUNK
    add_h = [None] * _NCHUNK

    def fire_feat(c):
      feat_h[c] = pltpu.async_copy(
          feat_hbm.at[wid * _NCHUNK + c], bufs[c % _NBUF], fsems[c % _NBUF])

    def fire_add(c):
      feat_h[c].wait()
      add_h[c] = pltpu.async_copy(
          spm.at[idx_v.at[c]], bufs[c % _NBUF], asems[c % _NBUF],
          add=True)

    for c in range(_NBUF):
      fire_feat(c)
    rpt = _NCPAD // _NS
    stage_h = pltpu.async_copy(negc_hbm.at[pl.ds(sid * rpt, rpt)],
                               spm.at[pl.ds(sid * rpt, rpt)], ssem)
    pltpu.sync_copy(y_hbm.at[wid], idx_v)
    stage_h.wait()
    plsc.subcore_barrier()
    fire_add(0)

    accs = tuple(jnp.zeros((_L,), jnp.float32) for _ in range(_NVJ))
    for c in range(_NCHUNK):
      if c + 1 < _NCHUNK:
        fire_add(c + 1)
      add_h[c].wait()
      buf = bufs[c % _NBUF]

      def row_body(i, acc, buf=buf):
        new = list(acc)
        base = i * _UNROLL
        for r in range(_UNROLL):
          for j in range(_NVJ):
            d = buf[base + r, pl.ds(j * _L, _L)]
            new[j] = new[j] + d * d
        return tuple(new)

      accs = lax.fori_loop(0, _CH // _UNROLL, row_body, accs)
      if c + _NBUF < _NCHUNK:
        fire_feat(c + _NBUF)

    total = accs[0]
    for j in range(1, _NVJ):
      total = total + accs[j]
    acc_v[...] = total
    pltpu.sync_copy(acc_v, out_hbm.at[wid])

  return body(feat_r, y_r, negc)


def kernel(feat, logits, y, centers):
  del logits
  y_r = y.astype(jnp.int32).reshape(_NW, _NCHUNK, _CH)
  feat_r = feat.reshape(_NW * _NCHUNK, _CH, _D)
  negc = jnp.zeros((_NCPAD, _D), jnp.float32).at[:_NCLASS].set(-centers)
  partials = _sc_center_loss(feat_r, y_r, negc)
  return jnp.sum(partials) / 2.0 / _B
